# baseline (speedup 1.0000x reference)
import dataclasses
import functools

import jax
import jax.numpy as jnp
from jax import lax
from jax.experimental import pallas as pl
from jax.experimental.pallas import tpu as pltpu
from jax.experimental.pallas import tpu_sc as plsc

TEMP = 0.07
N = 4096
B = 4096
P = 8
NEG = 128
NC = 2
NS = 16
NW = NC * NS
B_PER_W = B // NW
L = 16


def _sc_ratios(ssm_flat, anchors, pos_flat, neg_flat):
    mesh = plsc.VectorSubcoreMesh(core_axis_name="c", subcore_axis_name="s")
    cp = pltpu.CompilerParams()
    if "needs_layout_passes" in pltpu.CompilerParams.__dataclass_fields__:
        cp = dataclasses.replace(cp, needs_layout_passes=False)

    @functools.partial(
        pl.kernel,
        mesh=mesh,
        compiler_params=cp,
        out_type=jax.ShapeDtypeStruct((2 * B * P,), jnp.float32),
        scratch_types=[
            pltpu.VMEM((B_PER_W + L,), jnp.int32),
            pltpu.VMEM((B_PER_W * P + L,), jnp.int32),
            pltpu.VMEM((B_PER_W * NEG,), jnp.int32),
            pltpu.VMEM((P + 1, NEG), jnp.int32),
            pltpu.VMEM((L,), jnp.int32),
            pltpu.VMEM((P + 1, NEG), jnp.float32),
            pltpu.VMEM((L,), jnp.float32),
            pltpu.VMEM((B_PER_W * L,), jnp.float32),
            pltpu.VMEM((B_PER_W * L,), jnp.float32),
            pltpu.VMEM((2 * B_PER_W * P,), jnp.float32),
            pltpu.SemaphoreType.DMA,
        ],
    )
    def sc_kernel(ssm_hbm, anc_hbm, pos_hbm, neg_hbm, out_hbm,
                  anchors_l, pos_l, neg_l, idx_neg, idx_pos, vals_neg,
                  vals_pos, negsum_all, pexp_all, ratio_l, sem):
        wid = lax.axis_index("s") * NC + lax.axis_index("c")
        base_b = wid * B_PER_W

        pltpu.sync_copy(anc_hbm.at[pl.ds(base_b, B_PER_W)],
                        anchors_l.at[pl.ds(0, B_PER_W)])
        pltpu.sync_copy(pos_hbm.at[pl.ds(base_b * P, B_PER_W * P)],
                        pos_l.at[pl.ds(0, B_PER_W * P)])
        pltpu.sync_copy(neg_hbm.at[pl.ds(base_b * NEG, B_PER_W * NEG)], neg_l)

        lane = lax.iota(jnp.int32, L)

        @pl.loop(0, B_PER_W)
        def _phase_a(b):
            av = anchors_l[pl.ds(b, L)]
            a = av[0]
            arow = a * N
            posv = pos_l[pl.ds(b * P, L)]
            negv = [neg_l[pl.ds(b * NEG + L * j, L)] for j in range(NEG // L)]
            for p in range(P):
                rrow = posv[p] * N
                for j in range(NEG // L):
                    idx_neg[p, pl.ds(L * j, L)] = rrow + negv[j]
            for j in range(NEG // L):
                idx_neg[P, pl.ds(L * j, L)] = arow + negv[j]
            q = plsc.load_gather(pos_l, [b * P + lax.bitwise_and(lane, P - 1)])
            idx_pos[...] = jnp.where(lane < P, arow + q, q * N + a)

            descs = [
                pltpu.async_copy(ssm_hbm.at[idx_neg.at[k]], vals_neg.at[k], sem)
                for k in range(P + 1)
            ]
            dpos = pltpu.async_copy(ssm_hbm.at[idx_pos], vals_pos, sem)
            for d in descs:
                d.wait()
            dpos.wait()

            nsv = jnp.zeros((L,), jnp.float32)
            for k in range(P + 1):
                acc = jnp.exp(vals_neg[k, pl.ds(0, L)] / TEMP)
                for j in range(1, NEG // L):
                    acc = acc + jnp.exp(vals_neg[k, pl.ds(L * j, L)] / TEMP)
                nsv = jnp.where(lane == k, jnp.sum(acc), nsv)
            negsum_all[pl.ds(b * L, L)] = nsv
            pexp_all[pl.ds(b * L, L)] = jnp.exp(vals_pos[...] / TEMP)

        @pl.loop(0, B_PER_W * P // L)
        def _phase_b(g):
            item = g * L + lane
            bloc = lax.shift_right_logical(item, 3)
            ploc = lax.bitwise_and(item, P - 1)
            pidx = bloc * L + ploc
            pv_f = plsc.load_gather(pexp_all, [pidx])
            ns_f = plsc.load_gather(negsum_all, [bloc * L + P])
            ratio_l[pl.ds(g * L, L)] = pv_f / (pv_f + ns_f + 1e-10)

            pv_r = plsc.load_gather(pexp_all, [pidx + P])
            ns_r = plsc.load_gather(negsum_all, [pidx])
            ratio_l[pl.ds(B_PER_W * P + g * L, L)] = pv_r / (pv_r + ns_r + 1e-10)

        pltpu.sync_copy(ratio_l.at[pl.ds(0, B_PER_W * P)],
                        out_hbm.at[pl.ds(base_b * P, B_PER_W * P)])
        pltpu.sync_copy(ratio_l.at[pl.ds(B_PER_W * P, B_PER_W * P)],
                        out_hbm.at[pl.ds(B * P + base_b * P, B_PER_W * P)])

    return sc_kernel(ssm_flat, anchors, pos_flat, neg_flat)


def _tc_loss(ratios):
    def body(r_ref, o_ref):
        r = r_ref[...]
        o_ref[0, 0] = -jnp.mean(jnp.log(r + 1e-10))

    out = pl.pallas_call(
        body,
        out_shape=jax.ShapeDtypeStruct((1, 1), jnp.float32),
        out_specs=pl.BlockSpec(memory_space=pltpu.SMEM),
    )(ratios.reshape(2 * B * P // 128, 128))
    return out[0, 0]


def kernel(ssms_list, anchors, positives, negatives, embeddings):
    del embeddings
    ssm_flat = ssms_list.reshape(N * N)
    ratios = _sc_ratios(ssm_flat, anchors, positives.reshape(B * P),
                        negatives.reshape(B * NEG))
    return _tc_loss(ratios)


# speedup vs baseline: 1.3875x; 1.3875x over previous
import dataclasses
import functools

import jax
import jax.numpy as jnp
from jax import lax
from jax.experimental import pallas as pl
from jax.experimental.pallas import tpu as pltpu
from jax.experimental.pallas import tpu_sc as plsc

TEMP = 0.07
N = 4096
B = 4096
P = 8
NEG = 128
NC = 2
NS = 16
NW = NC * NS
B_PER_W = B // NW
L = 16


def _sc_ratios(ssm_flat, anchors, pos_flat, neg_flat):
    mesh = plsc.VectorSubcoreMesh(core_axis_name="c", subcore_axis_name="s")
    cp = pltpu.CompilerParams()
    if "needs_layout_passes" in pltpu.CompilerParams.__dataclass_fields__:
        cp = dataclasses.replace(cp, needs_layout_passes=False)

    @functools.partial(
        pl.kernel,
        mesh=mesh,
        compiler_params=cp,
        out_type=jax.ShapeDtypeStruct((2 * B * P,), jnp.float32),
        scratch_types=[
            pltpu.VMEM((B_PER_W + L,), jnp.int32),
            pltpu.VMEM((B_PER_W * P + L,), jnp.int32),
            pltpu.VMEM((B_PER_W * NEG,), jnp.int32),
            pltpu.VMEM((P + 1, NEG), jnp.int32),
            pltpu.VMEM((P + 1, NEG), jnp.int32),
            pltpu.VMEM((2, L), jnp.int32),
            pltpu.VMEM((P + 1, NEG), jnp.float32),
            pltpu.VMEM((P + 1, NEG), jnp.float32),
            pltpu.VMEM((2, L), jnp.float32),
            pltpu.VMEM((B_PER_W * L,), jnp.float32),
            pltpu.VMEM((B_PER_W * L,), jnp.float32),
            pltpu.VMEM((2 * B_PER_W * P,), jnp.float32),
            pltpu.SemaphoreType.DMA,
            pltpu.SemaphoreType.DMA,
        ],
    )
    def sc_kernel(ssm_hbm, anc_hbm, pos_hbm, neg_hbm, out_hbm,
                  anchors_l, pos_l, neg_l, idx_neg_a, idx_neg_b, idx_pos,
                  vals_neg_a, vals_neg_b, vals_pos, negsum_all, pexp_all,
                  ratio_l, sem0, sem1):
        wid = lax.axis_index("s") * NC + lax.axis_index("c")
        base_b = wid * B_PER_W
        sems = (sem0, sem1)
        idx_negs = (idx_neg_a, idx_neg_b)
        vals_negs = (vals_neg_a, vals_neg_b)

        pltpu.sync_copy(anc_hbm.at[pl.ds(base_b, B_PER_W)],
                        anchors_l.at[pl.ds(0, B_PER_W)])
        pltpu.sync_copy(pos_hbm.at[pl.ds(base_b * P, B_PER_W * P)],
                        pos_l.at[pl.ds(0, B_PER_W * P)])
        pltpu.sync_copy(neg_hbm.at[pl.ds(base_b * NEG, B_PER_W * NEG)], neg_l)

        lane = lax.iota(jnp.int32, L)

        def build_issue(b, buf):
            a = anchors_l[pl.ds(b, L)][0]
            arow = a * N
            posv = pos_l[pl.ds(b * P, L)]
            negv = [neg_l[pl.ds(b * NEG + L * j, L)] for j in range(NEG // L)]
            for p in range(P):
                rrow = posv[p] * N
                for j in range(NEG // L):
                    idx_negs[buf][p, pl.ds(L * j, L)] = rrow + negv[j]
            for j in range(NEG // L):
                idx_negs[buf][P, pl.ds(L * j, L)] = arow + negv[j]
            q = plsc.load_gather(pos_l, [b * P + lax.bitwise_and(lane, P - 1)])
            idx_pos[buf] = jnp.where(lane < P, arow + q, q * N + a)
            for k in range(P + 1):
                pltpu.async_copy(ssm_hbm.at[idx_negs[buf].at[k]],
                                 vals_negs[buf].at[k], sems[buf])
            pltpu.async_copy(ssm_hbm.at[idx_pos.at[buf]], vals_pos.at[buf],
                             sems[buf])

        def wait_buf(buf):
            for k in range(P + 1):
                pltpu.make_async_copy(ssm_hbm.at[idx_negs[buf].at[k]],
                                      vals_negs[buf].at[k], sems[buf]).wait()
            pltpu.make_async_copy(ssm_hbm.at[idx_pos.at[buf]],
                                  vals_pos.at[buf], sems[buf]).wait()

        def compute(b, buf):
            nsv = jnp.zeros((L,), jnp.float32)
            for k in range(P + 1):
                acc = jnp.exp(vals_negs[buf][k, pl.ds(0, L)] / TEMP)
                for j in range(1, NEG // L):
                    acc = acc + jnp.exp(
                        vals_negs[buf][k, pl.ds(L * j, L)] / TEMP)
                nsv = jnp.where(lane == k, jnp.sum(acc), nsv)
            negsum_all[pl.ds(b * L, L)] = nsv
            pexp_all[pl.ds(b * L, L)] = jnp.exp(vals_pos[buf] / TEMP)

        build_issue(0, 0)

        @pl.loop(0, B_PER_W // 2)
        def _phase_a(h):
            b0 = 2 * h
            build_issue(b0 + 1, 1)
            wait_buf(0)
            compute(b0, 0)

            @pl.when(h < B_PER_W // 2 - 1)
            def _():
                build_issue(b0 + 2, 0)

            wait_buf(1)
            compute(b0 + 1, 1)

        @pl.loop(0, B_PER_W * P // L)
        def _phase_b(g):
            item = g * L + lane
            bloc = lax.shift_right_logical(item, 3)
            ploc = lax.bitwise_and(item, P - 1)
            pidx = bloc * L + ploc
            pv_f = plsc.load_gather(pexp_all, [pidx])
            ns_f = plsc.load_gather(negsum_all, [bloc * L + P])
            ratio_l[pl.ds(g * L, L)] = pv_f / (pv_f + ns_f + 1e-10)

            pv_r = plsc.load_gather(pexp_all, [pidx + P])
            ns_r = plsc.load_gather(negsum_all, [pidx])
            ratio_l[pl.ds(B_PER_W * P + g * L, L)] = pv_r / (pv_r + ns_r + 1e-10)

        pltpu.sync_copy(ratio_l.at[pl.ds(0, B_PER_W * P)],
                        out_hbm.at[pl.ds(base_b * P, B_PER_W * P)])
        pltpu.sync_copy(ratio_l.at[pl.ds(B_PER_W * P, B_PER_W * P)],
                        out_hbm.at[pl.ds(B * P + base_b * P, B_PER_W * P)])

    return sc_kernel(ssm_flat, anchors, pos_flat, neg_flat)


def _tc_loss(ratios):
    def body(r_ref, o_ref):
        r = r_ref[...]
        o_ref[0, 0] = -jnp.mean(jnp.log(r + 1e-10))

    out = pl.pallas_call(
        body,
        out_shape=jax.ShapeDtypeStruct((1, 1), jnp.float32),
        out_specs=pl.BlockSpec(memory_space=pltpu.SMEM),
    )(ratios.reshape(2 * B * P // 128, 128))
    return out[0, 0]


def kernel(ssms_list, anchors, positives, negatives, embeddings):
    del embeddings
    ssm_flat = ssms_list.reshape(N * N)
    ratios = _sc_ratios(ssm_flat, anchors, positives.reshape(B * P),
                        negatives.reshape(B * NEG))
    return _tc_loss(ratios)


# speedup vs baseline: 1.4147x; 1.0196x over previous
import dataclasses
import functools

import jax
import jax.numpy as jnp
from jax import lax
from jax.experimental import pallas as pl
from jax.experimental.pallas import tpu as pltpu
from jax.experimental.pallas import tpu_sc as plsc

TEMP = 0.07
N = 4096
B = 4096
P = 8
NEG = 128
NC = 2
NS = 16
NW = NC * NS
B_PER_W = B // NW
L = 16


def _tc_exp_relayout(ssms_list):
    RB = 256

    def body(x_ref, o_ref):
        x = x_ref[0]
        o_ref[...] = jnp.exp((x / TEMP).reshape(RB * N // 128, 128))

    out = pl.pallas_call(
        body,
        grid=(N // RB,),
        in_specs=[pl.BlockSpec((1, RB, N), lambda i: (0, i, 0))],
        out_specs=pl.BlockSpec((RB * N // 128, 128), lambda i: (i, 0)),
        out_shape=jax.ShapeDtypeStruct((N * N // 128, 128), jnp.float32),
    )(ssms_list)
    return out.reshape(N * N)


def _sc_ratios(ssm_flat, anchors, pos_flat, neg_flat):
    mesh = plsc.VectorSubcoreMesh(core_axis_name="c", subcore_axis_name="s")
    cp = pltpu.CompilerParams()
    if "needs_layout_passes" in pltpu.CompilerParams.__dataclass_fields__:
        cp = dataclasses.replace(cp, needs_layout_passes=False)

    @functools.partial(
        pl.kernel,
        mesh=mesh,
        compiler_params=cp,
        out_type=jax.ShapeDtypeStruct((2 * B * P,), jnp.float32),
        scratch_types=[
            pltpu.VMEM((B_PER_W + L,), jnp.int32),
            pltpu.VMEM((B_PER_W * P + L,), jnp.int32),
            pltpu.VMEM((B_PER_W * NEG,), jnp.int32),
            pltpu.VMEM((P + 1, NEG), jnp.int32),
            pltpu.VMEM((P + 1, NEG), jnp.int32),
            pltpu.VMEM((2, L), jnp.int32),
            pltpu.VMEM((P + 1, NEG), jnp.float32),
            pltpu.VMEM((P + 1, NEG), jnp.float32),
            pltpu.VMEM((2, L), jnp.float32),
            pltpu.VMEM((B_PER_W * L,), jnp.float32),
            pltpu.VMEM((B_PER_W * L,), jnp.float32),
            pltpu.VMEM((2 * B_PER_W * P,), jnp.float32),
            pltpu.SemaphoreType.DMA,
            pltpu.SemaphoreType.DMA,
        ],
    )
    def sc_kernel(ssm_hbm, anc_hbm, pos_hbm, neg_hbm, out_hbm,
                  anchors_l, pos_l, neg_l, idx_neg_a, idx_neg_b, idx_pos,
                  vals_neg_a, vals_neg_b, vals_pos, negsum_all, pexp_all,
                  ratio_l, sem0, sem1):
        wid = lax.axis_index("s") * NC + lax.axis_index("c")
        base_b = wid * B_PER_W
        sems = (sem0, sem1)
        idx_negs = (idx_neg_a, idx_neg_b)
        vals_negs = (vals_neg_a, vals_neg_b)

        pltpu.sync_copy(anc_hbm.at[pl.ds(base_b, B_PER_W)],
                        anchors_l.at[pl.ds(0, B_PER_W)])
        pltpu.sync_copy(pos_hbm.at[pl.ds(base_b * P, B_PER_W * P)],
                        pos_l.at[pl.ds(0, B_PER_W * P)])
        pltpu.sync_copy(neg_hbm.at[pl.ds(base_b * NEG, B_PER_W * NEG)], neg_l)

        lane = lax.iota(jnp.int32, L)

        def build_issue(b, buf):
            a = anchors_l[pl.ds(b, L)][0]
            arow = a * N
            posv = pos_l[pl.ds(b * P, L)]
            negv = [neg_l[pl.ds(b * NEG + L * j, L)] for j in range(NEG // L)]
            for p in range(P):
                rrow = posv[p] * N
                for j in range(NEG // L):
                    idx_negs[buf][p, pl.ds(L * j, L)] = rrow + negv[j]
            for j in range(NEG // L):
                idx_negs[buf][P, pl.ds(L * j, L)] = arow + negv[j]
            q = plsc.load_gather(pos_l, [b * P + lax.bitwise_and(lane, P - 1)])
            idx_pos[buf] = jnp.where(lane < P, arow + q, q * N + a)
            for k in range(P + 1):
                pltpu.async_copy(ssm_hbm.at[idx_negs[buf].at[k]],
                                 vals_negs[buf].at[k], sems[buf])
            pltpu.async_copy(ssm_hbm.at[idx_pos.at[buf]], vals_pos.at[buf],
                             sems[buf])

        def wait_buf(buf):
            for k in range(P + 1):
                pltpu.make_async_copy(ssm_hbm.at[idx_negs[buf].at[k]],
                                      vals_negs[buf].at[k], sems[buf]).wait()
            pltpu.make_async_copy(ssm_hbm.at[idx_pos.at[buf]],
                                  vals_pos.at[buf], sems[buf]).wait()

        def compute(b, buf):
            nsv = jnp.zeros((L,), jnp.float32)
            for k in range(P + 1):
                acc = vals_negs[buf][k, pl.ds(0, L)]
                for j in range(1, NEG // L):
                    acc = acc + vals_negs[buf][k, pl.ds(L * j, L)]
                nsv = jnp.where(lane == k, jnp.sum(acc), nsv)
            negsum_all[pl.ds(b * L, L)] = nsv
            pexp_all[pl.ds(b * L, L)] = vals_pos[buf]

        build_issue(0, 0)

        @pl.loop(0, B_PER_W // 2)
        def _phase_a(h):
            b0 = 2 * h
            build_issue(b0 + 1, 1)
            wait_buf(0)
            compute(b0, 0)

            @pl.when(h < B_PER_W // 2 - 1)
            def _():
                build_issue(b0 + 2, 0)

            wait_buf(1)
            compute(b0 + 1, 1)

        @pl.loop(0, B_PER_W * P // L)
        def _phase_b(g):
            item = g * L + lane
            bloc = lax.shift_right_logical(item, 3)
            ploc = lax.bitwise_and(item, P - 1)
            pidx = bloc * L + ploc
            pv_f = plsc.load_gather(pexp_all, [pidx])
            ns_f = plsc.load_gather(negsum_all, [bloc * L + P])
            ratio_l[pl.ds(g * L, L)] = pv_f / (pv_f + ns_f + 1e-10)

            pv_r = plsc.load_gather(pexp_all, [pidx + P])
            ns_r = plsc.load_gather(negsum_all, [pidx])
            ratio_l[pl.ds(B_PER_W * P + g * L, L)] = pv_r / (pv_r + ns_r + 1e-10)

        pltpu.sync_copy(ratio_l.at[pl.ds(0, B_PER_W * P)],
                        out_hbm.at[pl.ds(base_b * P, B_PER_W * P)])
        pltpu.sync_copy(ratio_l.at[pl.ds(B_PER_W * P, B_PER_W * P)],
                        out_hbm.at[pl.ds(B * P + base_b * P, B_PER_W * P)])

    return sc_kernel(ssm_flat, anchors, pos_flat, neg_flat)


def _tc_loss(ratios):
    def body(r_ref, o_ref):
        r = r_ref[...]
        o_ref[0, 0] = -jnp.mean(jnp.log(r + 1e-10))

    out = pl.pallas_call(
        body,
        out_shape=jax.ShapeDtypeStruct((1, 1), jnp.float32),
        out_specs=pl.BlockSpec(memory_space=pltpu.SMEM),
    )(ratios.reshape(2 * B * P // 128, 128))
    return out[0, 0]


def kernel(ssms_list, anchors, positives, negatives, embeddings):
    del embeddings
    e_flat = _tc_exp_relayout(ssms_list)
    ratios = _sc_ratios(e_flat, anchors, positives.reshape(B * P),
                        negatives.reshape(B * NEG))
    return _tc_loss(ratios)


# speedup vs baseline: 1.4175x; 1.0020x over previous
import dataclasses
import functools

import jax
import jax.numpy as jnp
from jax import lax
from jax.experimental import pallas as pl
from jax.experimental.pallas import tpu as pltpu
from jax.experimental.pallas import tpu_sc as plsc

TEMP = 0.07
N = 4096
B = 4096
P = 8
NEG = 128
NC = 2
NS = 16
NW = NC * NS
B_PER_W = B // NW
L = 16


def _tc_exp_relayout(ssms_list):
    RB = 256

    def body(x_ref, o_ref):
        x = x_ref[0]
        o_ref[...] = jnp.exp((x / TEMP).reshape(RB * N // 128, 128))

    out = pl.pallas_call(
        body,
        grid=(N // RB,),
        in_specs=[pl.BlockSpec((1, RB, N), lambda i: (0, i, 0))],
        out_specs=pl.BlockSpec((RB * N // 128, 128), lambda i: (i, 0)),
        out_shape=jax.ShapeDtypeStruct((N * N // 128, 128), jnp.float32),
    )(ssms_list)
    return out.reshape(N * N)


def _sc_ratios(ssm_flat, anchors, pos_flat, neg_flat):
    mesh = plsc.VectorSubcoreMesh(core_axis_name="c", subcore_axis_name="s")
    cp = pltpu.CompilerParams()
    if "needs_layout_passes" in pltpu.CompilerParams.__dataclass_fields__:
        cp = dataclasses.replace(cp, needs_layout_passes=False)

    @functools.partial(
        pl.kernel,
        mesh=mesh,
        compiler_params=cp,
        out_type=jax.ShapeDtypeStruct((2 * B * P,), jnp.float32),
        scratch_types=[
            pltpu.VMEM((B_PER_W + L,), jnp.int32),
            pltpu.VMEM((B_PER_W * P + L,), jnp.int32),
            pltpu.VMEM((B_PER_W * NEG,), jnp.int32),
            pltpu.VMEM((P + 1, NEG), jnp.int32),
            pltpu.VMEM((P + 1, NEG), jnp.int32),
            pltpu.VMEM((P + 1, NEG), jnp.int32),
            pltpu.VMEM((P + 1, NEG), jnp.int32),
            pltpu.VMEM((4, L), jnp.int32),
            pltpu.VMEM(((P + 1) * NEG,), jnp.float32),
            pltpu.VMEM(((P + 1) * NEG,), jnp.float32),
            pltpu.VMEM(((P + 1) * NEG,), jnp.float32),
            pltpu.VMEM(((P + 1) * NEG,), jnp.float32),
            pltpu.VMEM((4, L), jnp.float32),
            pltpu.VMEM((B_PER_W * L,), jnp.float32),
            pltpu.VMEM((B_PER_W * L,), jnp.float32),
            pltpu.VMEM((2 * B_PER_W * P,), jnp.float32),
            pltpu.SemaphoreType.DMA,
            pltpu.SemaphoreType.DMA,
            pltpu.SemaphoreType.DMA,
            pltpu.SemaphoreType.DMA,
        ],
    )
    def sc_kernel(ssm_hbm, anc_hbm, pos_hbm, neg_hbm, out_hbm,
                  anchors_l, pos_l, neg_l, idx_neg_0, idx_neg_1, idx_neg_2,
                  idx_neg_3, idx_pos, vals_neg_0, vals_neg_1, vals_neg_2,
                  vals_neg_3, vals_pos, negsum_all, pexp_all, ratio_l,
                  sem0, sem1, sem2, sem3):
        wid = lax.axis_index("s") * NC + lax.axis_index("c")
        base_b = wid * B_PER_W
        sems = (sem0, sem1, sem2, sem3)
        idx_negs = (idx_neg_0, idx_neg_1, idx_neg_2, idx_neg_3)
        vals_negs = (vals_neg_0, vals_neg_1, vals_neg_2, vals_neg_3)

        pltpu.sync_copy(anc_hbm.at[pl.ds(base_b, B_PER_W)],
                        anchors_l.at[pl.ds(0, B_PER_W)])
        pltpu.sync_copy(pos_hbm.at[pl.ds(base_b * P, B_PER_W * P)],
                        pos_l.at[pl.ds(0, B_PER_W * P)])
        pltpu.sync_copy(neg_hbm.at[pl.ds(base_b * NEG, B_PER_W * NEG)], neg_l)

        lane = lax.iota(jnp.int32, L)

        def build_issue(b, buf):
            a = anchors_l[pl.ds(b, L)][0]
            arow = a * N
            posv = pos_l[pl.ds(b * P, L)]
            negv = [neg_l[pl.ds(b * NEG + L * j, L)] for j in range(NEG // L)]
            for p in range(P):
                rrow = posv[p] * N
                for j in range(NEG // L):
                    idx_negs[buf][p, pl.ds(L * j, L)] = rrow + negv[j]
            for j in range(NEG // L):
                idx_negs[buf][P, pl.ds(L * j, L)] = arow + negv[j]
            q = plsc.load_gather(pos_l, [b * P + lax.bitwise_and(lane, P - 1)])
            idx_pos[buf] = jnp.where(lane < P, arow + q, q * N + a)
            for k in range(P + 1):
                pltpu.async_copy(ssm_hbm.at[idx_negs[buf].at[k]],
                                 vals_negs[buf].at[pl.ds(k * NEG, NEG)],
                                 sems[buf])
            pltpu.async_copy(ssm_hbm.at[idx_pos.at[buf]], vals_pos.at[buf],
                             sems[buf])

        def wait_buf(buf):
            pltpu.make_async_copy(ssm_hbm.at[pl.ds(0, (P + 1) * NEG)],
                                  vals_negs[buf], sems[buf]).wait()
            pltpu.make_async_copy(ssm_hbm.at[idx_pos.at[buf]],
                                  vals_pos.at[buf], sems[buf]).wait()

        def compute(b, buf):
            nsv = jnp.zeros((L,), jnp.float32)
            for k in range(P + 1):
                acc = vals_negs[buf][pl.ds(k * NEG, L)]
                for j in range(1, NEG // L):
                    acc = acc + vals_negs[buf][pl.ds(k * NEG + L * j, L)]
                nsv = jnp.where(lane == k, jnp.sum(acc), nsv)
            negsum_all[pl.ds(b * L, L)] = nsv
            pexp_all[pl.ds(b * L, L)] = vals_pos[buf]

        build_issue(0, 0)
        build_issue(1, 1)
        build_issue(2, 2)

        @pl.loop(0, (B_PER_W - 4) // 4)
        def _phase_a(h):
            b0 = 4 * h
            for u in range(4):
                wait_buf(u)
                build_issue(b0 + 3 + u, (u + 3) % 4)
                compute(b0 + u, u)

        b0 = B_PER_W - 4
        build_issue(B_PER_W - 1, 3)
        for u in range(4):
            wait_buf(u)
            compute(b0 + u, u)

        @pl.loop(0, B_PER_W * P // L)
        def _phase_b(g):
            item = g * L + lane
            bloc = lax.shift_right_logical(item, 3)
            ploc = lax.bitwise_and(item, P - 1)
            pidx = bloc * L + ploc
            pv_f = plsc.load_gather(pexp_all, [pidx])
            ns_f = plsc.load_gather(negsum_all, [bloc * L + P])
            ratio_l[pl.ds(g * L, L)] = pv_f / (pv_f + ns_f + 1e-10)

            pv_r = plsc.load_gather(pexp_all, [pidx + P])
            ns_r = plsc.load_gather(negsum_all, [pidx])
            ratio_l[pl.ds(B_PER_W * P + g * L, L)] = pv_r / (pv_r + ns_r + 1e-10)

        pltpu.sync_copy(ratio_l.at[pl.ds(0, B_PER_W * P)],
                        out_hbm.at[pl.ds(base_b * P, B_PER_W * P)])
        pltpu.sync_copy(ratio_l.at[pl.ds(B_PER_W * P, B_PER_W * P)],
                        out_hbm.at[pl.ds(B * P + base_b * P, B_PER_W * P)])

    return sc_kernel(ssm_flat, anchors, pos_flat, neg_flat)


def _tc_loss(ratios):
    def body(r_ref, o_ref):
        r = r_ref[...]
        o_ref[0, 0] = -jnp.mean(jnp.log(r + 1e-10))

    out = pl.pallas_call(
        body,
        out_shape=jax.ShapeDtypeStruct((1, 1), jnp.float32),
        out_specs=pl.BlockSpec(memory_space=pltpu.SMEM),
    )(ratios.reshape(2 * B * P // 128, 128))
    return out[0, 0]


def kernel(ssms_list, anchors, positives, negatives, embeddings):
    del embeddings
    e_flat = _tc_exp_relayout(ssms_list)
    ratios = _sc_ratios(e_flat, anchors, positives.reshape(B * P),
                        negatives.reshape(B * NEG))
    return _tc_loss(ratios)
